# Initial kernel scaffold; baseline (speedup 1.0000x reference)
import jax
import jax.numpy as jnp
from jax.experimental import pallas as pl


def kernel(x, es):
    raise NotImplementedError("write your pallas kernel here")


# speedup vs baseline: 13.1348x; 13.1348x over previous
import jax
import jax.numpy as jnp
from jax import lax
from jax.experimental import pallas as pl
from jax.experimental.pallas import tpu as pltpu
from jax.experimental.pallas import tpu_sc as plsc

N = 10000
E = 160000
D = 256
H = 128
NC = 2
NS = 16
C = 80
K = (E // NS) // C
G = 25
R = 32
NCH = N // R
REM = N - NCH * R
NPASS = -(-NCH // NS)
HB = 80


def _finalize_chunk(c, nrows, cid, x_in, out, accv, xv, cntv, acc_sh,
                    cnt_sh):
    nb = c * R
    pltpu.sync_copy(acc_sh.at[pl.ds(nb, nrows)], accv.at[pl.ds(0, nrows)])
    pltpu.sync_copy(x_in.at[pl.ds(nb, nrows), pl.ds(cid * H, H)],
                    xv.at[pl.ds(0, nrows)])
    pltpu.sync_copy(cnt_sh.at[lax.shift_right_logical(nb, 7), 0], cntv)

    lanes0 = jnp.zeros((16,), jnp.int32)
    coff = nb & (H - 1)

    def fin(i, _):
        cnt = plsc.load_gather(cntv, [lanes0 + (coff + i)])
        rden = 1.0 / jnp.maximum(cnt, 1.0)
        mask = cnt > 0.5
        for j in range(H // 16):
            accv[i, pl.ds(j * 16, 16)] = accv[i, pl.ds(j * 16, 16)] * rden
            xx = xv[i, pl.ds(j * 16, 16)]
            xv[i, pl.ds(j * 16, 16)] = jnp.where(mask, xx, 0.0)
        return _
    lax.fori_loop(0, nrows, fin, 0)

    pltpu.sync_copy(accv.at[pl.ds(0, nrows)],
                    out.at[pl.ds(nb, nrows), pl.ds(cid * H, H)])
    pltpu.sync_copy(xv.at[pl.ds(0, nrows)],
                    out.at[pl.ds(nb, nrows), pl.ds(D + cid * H, H)])


def _body(xt, x_in, rows2, col2, out, ridx, cidx, rowsb, accv, xv,
          cntb, cntv, idv, acc_sh, cnt_sh, sems):
    cid = lax.axis_index("c")
    sid = lax.axis_index("s")

    def zacc(i, _):
        for j in range(H // 16):
            accv[i, pl.ds(j * 16, 16)] = jnp.zeros((16,), jnp.float32)
        return _
    lax.fori_loop(0, R, zacc, 0)

    def zcnt(i, _):
        for j in range(H // 16):
            cntb[i, 0, pl.ds(j * 16, 16)] = jnp.zeros((16,), jnp.float32)
        return _
    lax.fori_loop(0, HB, zcnt, 0)

    for j in range(HB // 16):
        idv[pl.ds(j * 16, 16)] = lax.iota(jnp.int32, 16) + j * 16

    for q in range(NPASS):
        c = q * NS + sid

        @pl.when(c < NCH)
        def _():
            pltpu.sync_copy(accv, acc_sh.at[pl.ds(c * R, R)])

    @pl.when(sid == NS - 1)
    def _():
        pltpu.sync_copy(accv.at[pl.ds(0, REM)],
                        acc_sh.at[pl.ds(NCH * R, REM)])

    @pl.when(sid == 0)
    def _():
        pltpu.sync_copy(cntb, cnt_sh)

    pcnt, _pm = plsc.scan_count(idv[pl.ds(0, 16)])
    bias = 1 - jnp.max(pcnt)

    plsc.subcore_barrier()

    for g in range(K // G):
        pltpu.sync_copy(rows2.at[cid, sid, pl.ds(g * G, G)], ridx)
        pltpu.sync_copy(col2.at[sid, pl.ds(g * G, G)], cidx)
        pltpu.async_copy(xt.at[ridx.at[0, 0]], rowsb.at[0], sems.at[0])

        def step(k, carry):
            @pl.when(k < G - 1)
            def _():
                b = (k + 1) & 1
                pltpu.async_copy(xt.at[ridx.at[k + 1, 0]], rowsb.at[b],
                                 sems.at[b])

            for j in range(C // 16):
                cv = cidx[k, 0, pl.ds(j * 16, 16)]
                cnt, last = plsc.scan_count(cv)
                val = (cnt + bias).astype(jnp.float32)
                plsc.addupdate_scatter(
                    cntb, [lax.shift_right_logical(cv, 7),
                           jnp.zeros((16,), jnp.int32),
                           cv & (H - 1)], val, mask=last)

            b = k & 1
            pltpu.make_async_copy(xt.at[ridx.at[k, 0]], rowsb.at[b],
                                  sems.at[b]).wait()
            pltpu.sync_copy(rowsb.at[b], acc_sh.at[cidx.at[k, 0]],
                            add=True)
            return carry
        lax.fori_loop(0, G, step, 0)

    pltpu.sync_copy(cntb, cnt_sh.at[idv], add=True)

    plsc.subcore_barrier()

    for q in range(NPASS):
        c = q * NS + sid

        @pl.when(c < NCH)
        def _():
            _finalize_chunk(c, R, cid, x_in, out, accv, xv, cntv,
                            acc_sh, cnt_sh)

    @pl.when(sid == NS - 1)
    def _():
        _finalize_chunk(NCH, REM, cid, x_in, out, accv, xv, cntv,
                        acc_sh, cnt_sh)


@jax.jit
def kernel(x, es):
    es = es.astype(jnp.int32)
    col = es[0]
    row = es[1]
    xt = x.reshape(2 * N, H)
    r2 = 2 * row
    rows2 = jnp.stack([r2, r2 + 1]).reshape(NC, NS, K, 1, C)
    col2 = col.reshape(NS, K, 1, C)

    mesh = plsc.VectorSubcoreMesh(core_axis_name="c", subcore_axis_name="s")
    f = pl.kernel(
        _body,
        out_type=jax.ShapeDtypeStruct((N, 2 * D), jnp.float32),
        mesh=mesh,
        compiler_params=pltpu.CompilerParams(needs_layout_passes=False),
        scratch_types=[
            pltpu.VMEM((G, 1, C), jnp.int32),
            pltpu.VMEM((G, 1, C), jnp.int32),
            pltpu.VMEM((2, C, H), jnp.float32),
            pltpu.VMEM((R, H), jnp.float32),
            pltpu.VMEM((R, H), jnp.float32),
            pltpu.VMEM((HB, 1, H), jnp.float32),
            pltpu.VMEM((H,), jnp.float32),
            pltpu.VMEM((HB,), jnp.int32),
            pltpu.VMEM_SHARED((N, H), jnp.float32),
            pltpu.VMEM_SHARED((HB, 1, H), jnp.float32),
            pltpu.SemaphoreType.DMA((2,)),
        ],
    )
    return f(xt, x, rows2, col2)
